# speedup vs baseline: 1.7683x; 1.7683x over previous
import functools

import jax
import jax.numpy as jnp
from jax import lax
from jax.experimental import pallas as pl
from jax.experimental.pallas import tpu as pltpu
from jax.experimental.pallas import tpu_sc as plsc

EMBED = 2048
NC = 2
NS = 16
NW = NC * NS
CHUNK = 16


def _emb_body(n_chunks, idx_hbm, table_hbm, out_hbm, idx_v, buf0, buf1,
              sem_g0, sem_g1, sem_o0, sem_o1):
  cid = lax.axis_index("c")
  sid = lax.axis_index("s")
  wid = sid * NC + cid
  base = wid * (n_chunks * CHUNK)

  pltpu.sync_copy(idx_hbm.at[wid], idx_v)

  pltpu.async_copy(table_hbm.at[idx_v.at[0]], buf0, sem_g0)
  pltpu.async_copy(table_hbm.at[idx_v.at[1]], buf1, sem_g1)

  @pl.loop(0, n_chunks, step=2)
  def _(g):
    for b, (buf, sem_g, sem_o) in enumerate(
        ((buf0, sem_g0, sem_o0), (buf1, sem_g1, sem_o1))):
      i = g + b
      pltpu.make_async_copy(table_hbm.at[idx_v.at[i]], buf, sem_g).wait()
      cp_out = pltpu.async_copy(
          buf, out_hbm.at[pl.ds(base + i * CHUNK, CHUNK)], sem_o)

      cp_out.wait()

      @pl.when(i + 2 < n_chunks)
      def _():
        pltpu.async_copy(table_hbm.at[idx_v.at[i + 2]], buf, sem_g)


def kernel(input_ids, embed_table):
  batch, seq = input_ids.shape
  total = batch * seq
  n_chunks = total // (NW * CHUNK)
  idx = input_ids.reshape(NW, n_chunks, CHUNK).astype(jnp.int32)

  mesh = plsc.VectorSubcoreMesh(core_axis_name="c", subcore_axis_name="s")
  k = pl.kernel(
      functools.partial(_emb_body, n_chunks),
      out_type=jax.ShapeDtypeStruct((total, EMBED), jnp.float32),
      mesh=mesh,
      scratch_types=[
          pltpu.VMEM((n_chunks, CHUNK), jnp.int32),
          pltpu.VMEM((CHUNK, EMBED), jnp.float32),
          pltpu.VMEM((CHUNK, EMBED), jnp.float32),
          pltpu.SemaphoreType.DMA,
          pltpu.SemaphoreType.DMA,
          pltpu.SemaphoreType.DMA,
          pltpu.SemaphoreType.DMA,
      ],
  )
  out = k(idx, embed_table)
  return out.reshape(batch, seq, EMBED)
